# baseline (speedup 1.0000x reference)
import functools

import jax
import jax.numpy as jnp
from jax import lax
from jax.experimental import pallas as pl
from jax.experimental.pallas import tpu as pltpu
from jax.experimental.pallas import tpu_sc as plsc

_NC = 2
_NS = 16
_NW = _NC * _NS
_CH = 64
_G = 26

_DN = (((1,), (1,)), ((), ()))


def _edge_embed(edge_attr, We, be):
    E, DE = edge_attr.shape
    D = We.shape[0]
    BE = 8000
    assert E % BE == 0

    def body(ea, w, b, o):
        o[...] = jnp.maximum(
            lax.dot_general(ea[...], w[...], _DN,
                            preferred_element_type=jnp.float32) + b[...],
            0.0)

    return pl.pallas_call(
        body,
        grid=(E // BE,),
        in_specs=[
            pl.BlockSpec((BE, DE), lambda i: (i, 0)),
            pl.BlockSpec((D, DE), lambda i: (0, 0)),
            pl.BlockSpec((1, D), lambda i: (0, 0)),
        ],
        out_specs=pl.BlockSpec((BE, D), lambda i: (i, 0)),
        out_shape=jax.ShapeDtypeStruct((E, D), jnp.float32),
    )(edge_attr, We, be.reshape(1, D))


def _sc_aggregate(x, ef, idx_hbm):
    N, D = x.shape
    E = ef.shape[0]
    nch = E // _CH
    assert nch * _CH == E
    bcnt = nch // _NW
    extra = nch - bcnt * _NW
    ngrp = bcnt // _G
    assert ngrp * _G == bcnt
    nb = N // _CH
    nbr = N - nb * _CH
    nbpt = -(-nb // _NS)

    mesh = plsc.VectorSubcoreMesh(core_axis_name="c", subcore_axis_name="s")

    @functools.partial(
        pl.kernel,
        out_type=jax.ShapeDtypeStruct((_NC, N, D), jnp.float32),
        mesh=mesh,
        scratch_types=[
            pltpu.VMEM((_G, 1, _CH), jnp.int32),
            pltpu.VMEM((_G, 1, _CH), jnp.int32),
            pltpu.VMEM((2, _CH, D), jnp.float32),
            pltpu.VMEM((2, _CH, D), jnp.float32),
            pltpu.VMEM_SHARED((N, D), jnp.float32),
            pltpu.SemaphoreType.DMA((2,)),
            pltpu.SemaphoreType.DMA((2,)),
            pltpu.SemaphoreType.DMA((2,)),
            pltpu.SemaphoreType.DMA((2,)),
    ])
    def k(x_hbm, ef_hbm, idx, out_hbm,
          src_v, dst_v, xrows, efrows, agg_sh, semg, seme, semsx, semse):
        c = lax.axis_index("c")
        s = lax.axis_index("s")
        wid = s * _NC + c
        c0 = bcnt * wid + jnp.minimum(wid, extra)

        def zr(i, carry):
            xrows[0, i // (D // 16), pl.ds((i % (D // 16)) * 16, 16)] = (
                jnp.zeros((16,), jnp.float32))
            return carry
        lax.fori_loop(0, _CH * (D // 16), zr, 0)
        for t in range(nbpt):
            b = s + t * _NS

            @pl.when(b < nb)
            def _():
                pltpu.sync_copy(xrows.at[0], agg_sh.at[pl.ds(b * _CH, _CH)])
        if nbr:
            @pl.when(s == nb % _NS)
            def _():
                pltpu.sync_copy(xrows.at[0, pl.ds(0, nbr)],
                                agg_sh.at[pl.ds(nb * _CH, nbr)])
        plsc.subcore_barrier()

        def issue(ch, j, b):
            pltpu.async_copy(x_hbm.at[src_v.at[j, 0]], xrows.at[b],
                             semg.at[b])
            pltpu.async_copy(ef_hbm.at[pl.ds(ch * _CH, _CH)], efrows.at[b],
                             seme.at[b])

        def wait_in(j, b):
            pltpu.make_async_copy(x_hbm.at[src_v.at[j, 0]], xrows.at[b],
                                  semg.at[b]).wait()
            pltpu.make_async_copy(ef_hbm.at[pl.ds(0, _CH)], efrows.at[b],
                                  seme.at[b]).wait()

        def scat(j, b):
            pltpu.async_copy(xrows.at[b], agg_sh.at[dst_v.at[j, 0]],
                             semsx.at[b], add=True)
            pltpu.async_copy(efrows.at[b], agg_sh.at[dst_v.at[j, 0]],
                             semse.at[b], add=True)

        def wait_scat(j, b):
            pltpu.make_async_copy(xrows.at[b], agg_sh.at[dst_v.at[j, 0]],
                                  semsx.at[b]).wait()
            pltpu.make_async_copy(efrows.at[b], agg_sh.at[dst_v.at[j, 0]],
                                  semse.at[b]).wait()

        def group(g, carry):
            g0 = c0 + g * _G
            pltpu.sync_copy(idx.at[0, pl.ds(g0, _G)], src_v)
            pltpu.sync_copy(idx.at[1, pl.ds(g0, _G)], dst_v)
            issue(g0, 0, 0)

            def chunk(j, carry2):
                p = lax.rem(j, 2)
                q = 1 - p

                @pl.when(j + 1 < _G)
                def _():
                    @pl.when(j >= 1)
                    def _():
                        wait_scat(j - 1, q)
                    issue(g0 + j + 1, j + 1, q)
                wait_in(j, p)
                scat(j, p)
                return carry2
            lax.fori_loop(0, _G, chunk, 0)
            wait_scat(_G - 2, lax.rem(_G, 2))
            wait_scat(_G - 1, lax.rem(_G - 1, 2))
            return carry
        lax.fori_loop(0, ngrp, group, 0)

        if extra:
            @pl.when(wid < extra)
            def _():
                ce = c0 + bcnt
                pltpu.sync_copy(idx.at[0, pl.ds(ce, 1)],
                                src_v.at[pl.ds(0, 1)])
                pltpu.sync_copy(idx.at[1, pl.ds(ce, 1)],
                                dst_v.at[pl.ds(0, 1)])
                pltpu.sync_copy(x_hbm.at[src_v.at[0, 0]], xrows.at[0])
                pltpu.sync_copy(ef_hbm.at[pl.ds(ce * _CH, _CH)], efrows.at[0])
                pltpu.sync_copy(xrows.at[0], agg_sh.at[dst_v.at[0, 0]],
                                add=True)
                pltpu.sync_copy(efrows.at[0], agg_sh.at[dst_v.at[0, 0]],
                                add=True)

        plsc.subcore_barrier()
        for t in range(nbpt):
            b = s + t * _NS

            @pl.when(b < nb)
            def _():
                pltpu.sync_copy(agg_sh.at[pl.ds(b * _CH, _CH)],
                                out_hbm.at[c, pl.ds(b * _CH, _CH)])
        if nbr:
            @pl.when(s == nb % _NS)
            def _():
                pltpu.sync_copy(agg_sh.at[pl.ds(nb * _CH, nbr)],
                                out_hbm.at[c, pl.ds(nb * _CH, nbr)])

    return k(x, ef, idx_hbm)


def _mlp(x, agg, eps, W1, b1, g1, bt1, W2, b2, g2, bt2):
    N, D = x.shape

    def body(eps_ref, x_ref, a_ref, w1, b1r, g1r, t1r,
             w2, b2r, g2r, t2r, o):
        out = (1.0 + eps_ref[0, 0]) * x_ref[...] + a_ref[0] + a_ref[1]
        h = lax.dot_general(out, w1[...], _DN,
                            preferred_element_type=jnp.float32) + b1r[...]
        mu = jnp.mean(h, axis=0, keepdims=True)
        var = jnp.mean((h - mu) ** 2, axis=0, keepdims=True)
        h = jnp.maximum((h - mu) / jnp.sqrt(var + 1e-5) * g1r[...] + t1r[...],
                        0.0)
        h = lax.dot_general(h, w2[...], _DN,
                            preferred_element_type=jnp.float32) + b2r[...]
        mu = jnp.mean(h, axis=0, keepdims=True)
        var = jnp.mean((h - mu) ** 2, axis=0, keepdims=True)
        o[...] = jnp.maximum(
            (h - mu) / jnp.sqrt(var + 1e-5) * g2r[...] + t2r[...], 0.0)

    vspec = pl.BlockSpec(memory_space=pltpu.VMEM)
    return pl.pallas_call(
        body,
        in_specs=[pl.BlockSpec(memory_space=pltpu.SMEM)] + [vspec] * 10,
        out_specs=vspec,
        out_shape=jax.ShapeDtypeStruct((N, D), jnp.float32),
    )(eps.reshape(1, 1), x, agg, W1,
      b1.reshape(1, D), g1.reshape(1, D), bt1.reshape(1, D), W2,
      b2.reshape(1, D), g2.reshape(1, D), bt2.reshape(1, D))


def kernel(x, edge_index, edge_attr, epsilon, We, be,
           W1, b1, g1, bt1, W2, b2, g2, bt2):
    E = edge_index.shape[1]
    idx4 = edge_index.reshape(2, E // _CH, 1, _CH)
    ef = _edge_embed(edge_attr, We, be)
    agg = _sc_aggregate(x, ef, idx4)
    return _mlp(x, agg, epsilon, W1, b1, g1, bt1, W2, b2, g2, bt2)


# speedup vs baseline: 1.0057x; 1.0057x over previous
import functools

import jax
import jax.numpy as jnp
from jax import lax
from jax.experimental import pallas as pl
from jax.experimental.pallas import tpu as pltpu
from jax.experimental.pallas import tpu_sc as plsc

_NC = 2
_NS = 16
_NW = _NC * _NS
_CH = 64
_G = 26

_DN = (((1,), (1,)), ((), ()))


def _edge_embed(edge_attr, We, be):
    E, DE = edge_attr.shape
    D = We.shape[0]
    BE = 16000
    assert E % BE == 0

    def body(ea, w, b, o):
        o[...] = jnp.maximum(
            lax.dot_general(ea[...], w[...], _DN,
                            preferred_element_type=jnp.float32) + b[...],
            0.0)

    return pl.pallas_call(
        body,
        grid=(E // BE,),
        in_specs=[
            pl.BlockSpec((BE, DE), lambda i: (i, 0)),
            pl.BlockSpec((D, DE), lambda i: (0, 0)),
            pl.BlockSpec((1, D), lambda i: (0, 0)),
        ],
        out_specs=pl.BlockSpec((BE, D), lambda i: (i, 0)),
        out_shape=jax.ShapeDtypeStruct((E, D), jnp.float32),
    )(edge_attr, We, be.reshape(1, D))


def _sc_aggregate(x, ef, idx_hbm):
    N, D = x.shape
    E = ef.shape[0]
    nch = E // _CH
    assert nch * _CH == E
    bcnt = nch // _NW
    extra = nch - bcnt * _NW
    ngrp = bcnt // _G
    assert ngrp * _G == bcnt
    nb = N // _CH
    nbr = N - nb * _CH
    nbpt = -(-nb // _NS)

    mesh = plsc.VectorSubcoreMesh(core_axis_name="c", subcore_axis_name="s")

    @functools.partial(
        pl.kernel,
        out_type=jax.ShapeDtypeStruct((_NC, N, D), jnp.float32),
        mesh=mesh,
        scratch_types=[
            pltpu.VMEM((_G, 1, _CH), jnp.int32),
            pltpu.VMEM((_G, 1, _CH), jnp.int32),
            pltpu.VMEM((2, _CH, D), jnp.float32),
            pltpu.VMEM((2, _CH, D), jnp.float32),
            pltpu.VMEM_SHARED((N, D), jnp.float32),
            pltpu.SemaphoreType.DMA((2,)),
            pltpu.SemaphoreType.DMA((2,)),
            pltpu.SemaphoreType.DMA((2,)),
            pltpu.SemaphoreType.DMA((2,)),
    ])
    def k(x_hbm, ef_hbm, idx, out_hbm,
          src_v, dst_v, xrows, efrows, agg_sh, semg, seme, semsx, semse):
        c = lax.axis_index("c")
        s = lax.axis_index("s")
        wid = s * _NC + c
        c0 = bcnt * wid + jnp.minimum(wid, extra)

        def zr(i, carry):
            xrows[0, i // (D // 16), pl.ds((i % (D // 16)) * 16, 16)] = (
                jnp.zeros((16,), jnp.float32))
            return carry
        lax.fori_loop(0, _CH * (D // 16), zr, 0)
        for t in range(nbpt):
            b = s + t * _NS

            @pl.when(b < nb)
            def _():
                pltpu.sync_copy(xrows.at[0], agg_sh.at[pl.ds(b * _CH, _CH)])
        if nbr:
            @pl.when(s == nb % _NS)
            def _():
                pltpu.sync_copy(xrows.at[0, pl.ds(0, nbr)],
                                agg_sh.at[pl.ds(nb * _CH, nbr)])
        plsc.subcore_barrier()

        def issue(ch, j, b):
            pltpu.async_copy(x_hbm.at[src_v.at[j, 0]], xrows.at[b],
                             semg.at[b])
            pltpu.async_copy(ef_hbm.at[pl.ds(ch * _CH, _CH)], efrows.at[b],
                             seme.at[b])

        def wait_in(j, b):
            pltpu.make_async_copy(x_hbm.at[src_v.at[j, 0]], xrows.at[b],
                                  semg.at[b]).wait()
            pltpu.make_async_copy(ef_hbm.at[pl.ds(0, _CH)], efrows.at[b],
                                  seme.at[b]).wait()

        def scat(j, b):
            pltpu.async_copy(xrows.at[b], agg_sh.at[dst_v.at[j, 0]],
                             semsx.at[b], add=True)
            pltpu.async_copy(efrows.at[b], agg_sh.at[dst_v.at[j, 0]],
                             semse.at[b], add=True)

        def wait_scat(j, b):
            pltpu.make_async_copy(xrows.at[b], agg_sh.at[dst_v.at[j, 0]],
                                  semsx.at[b]).wait()
            pltpu.make_async_copy(efrows.at[b], agg_sh.at[dst_v.at[j, 0]],
                                  semse.at[b]).wait()

        def group(g, carry):
            g0 = c0 + g * _G
            pltpu.sync_copy(idx.at[0, pl.ds(g0, _G)], src_v)
            pltpu.sync_copy(idx.at[1, pl.ds(g0, _G)], dst_v)
            issue(g0, 0, 0)

            def chunk(j, carry2):
                p = lax.rem(j, 2)
                q = 1 - p

                @pl.when(j + 1 < _G)
                def _():
                    @pl.when(j >= 1)
                    def _():
                        wait_scat(j - 1, q)
                    issue(g0 + j + 1, j + 1, q)
                wait_in(j, p)
                scat(j, p)
                return carry2
            lax.fori_loop(0, _G, chunk, 0)
            wait_scat(_G - 2, lax.rem(_G, 2))
            wait_scat(_G - 1, lax.rem(_G - 1, 2))
            return carry
        lax.fori_loop(0, ngrp, group, 0)

        if extra:
            @pl.when(wid < extra)
            def _():
                ce = c0 + bcnt
                pltpu.sync_copy(idx.at[0, pl.ds(ce, 1)],
                                src_v.at[pl.ds(0, 1)])
                pltpu.sync_copy(idx.at[1, pl.ds(ce, 1)],
                                dst_v.at[pl.ds(0, 1)])
                pltpu.sync_copy(x_hbm.at[src_v.at[0, 0]], xrows.at[0])
                pltpu.sync_copy(ef_hbm.at[pl.ds(ce * _CH, _CH)], efrows.at[0])
                pltpu.sync_copy(xrows.at[0], agg_sh.at[dst_v.at[0, 0]],
                                add=True)
                pltpu.sync_copy(efrows.at[0], agg_sh.at[dst_v.at[0, 0]],
                                add=True)

        plsc.subcore_barrier()
        for t in range(nbpt):
            b = s + t * _NS

            @pl.when(b < nb)
            def _():
                pltpu.sync_copy(agg_sh.at[pl.ds(b * _CH, _CH)],
                                out_hbm.at[c, pl.ds(b * _CH, _CH)])
        if nbr:
            @pl.when(s == nb % _NS)
            def _():
                pltpu.sync_copy(agg_sh.at[pl.ds(nb * _CH, nbr)],
                                out_hbm.at[c, pl.ds(nb * _CH, nbr)])

    return k(x, ef, idx_hbm)


def _mlp(x, agg, eps, W1, b1, g1, bt1, W2, b2, g2, bt2):
    N, D = x.shape

    def body(eps_ref, x_ref, a_ref, w1, b1r, g1r, t1r,
             w2, b2r, g2r, t2r, o):
        out = (1.0 + eps_ref[0, 0]) * x_ref[...] + a_ref[0] + a_ref[1]
        h = lax.dot_general(out, w1[...], _DN,
                            preferred_element_type=jnp.float32) + b1r[...]
        mu = jnp.mean(h, axis=0, keepdims=True)
        var = jnp.mean((h - mu) ** 2, axis=0, keepdims=True)
        h = jnp.maximum((h - mu) / jnp.sqrt(var + 1e-5) * g1r[...] + t1r[...],
                        0.0)
        h = lax.dot_general(h, w2[...], _DN,
                            preferred_element_type=jnp.float32) + b2r[...]
        mu = jnp.mean(h, axis=0, keepdims=True)
        var = jnp.mean((h - mu) ** 2, axis=0, keepdims=True)
        o[...] = jnp.maximum(
            (h - mu) / jnp.sqrt(var + 1e-5) * g2r[...] + t2r[...], 0.0)

    vspec = pl.BlockSpec(memory_space=pltpu.VMEM)
    return pl.pallas_call(
        body,
        in_specs=[pl.BlockSpec(memory_space=pltpu.SMEM)] + [vspec] * 10,
        out_specs=vspec,
        out_shape=jax.ShapeDtypeStruct((N, D), jnp.float32),
    )(eps.reshape(1, 1), x, agg, W1,
      b1.reshape(1, D), g1.reshape(1, D), bt1.reshape(1, D), W2,
      b2.reshape(1, D), g2.reshape(1, D), bt2.reshape(1, D))


def kernel(x, edge_index, edge_attr, epsilon, We, be,
           W1, b1, g1, bt1, W2, b2, g2, bt2):
    E = edge_index.shape[1]
    idx4 = edge_index.reshape(2, E // _CH, 1, _CH)
    ef = _edge_embed(edge_attr, We, be)
    agg = _sc_aggregate(x, ef, idx4)
    return _mlp(x, agg, epsilon, W1, b1, g1, bt1, W2, b2, g2, bt2)


# speedup vs baseline: 1.0142x; 1.0084x over previous
import functools

import jax
import jax.numpy as jnp
from jax import lax
from jax.experimental import pallas as pl
from jax.experimental.pallas import tpu as pltpu
from jax.experimental.pallas import tpu_sc as plsc

_NC = 2
_NS = 16
_NW = _NC * _NS
_CH = 64
_G = 39

_DN = (((1,), (1,)), ((), ()))


def _edge_embed(edge_attr, We, be):
    E, DE = edge_attr.shape
    D = We.shape[0]
    BE = 16000
    assert E % BE == 0

    def body(ea, w, b, o):
        o[...] = jnp.maximum(
            lax.dot_general(ea[...], w[...], _DN,
                            preferred_element_type=jnp.float32) + b[...],
            0.0)

    return pl.pallas_call(
        body,
        grid=(E // BE,),
        in_specs=[
            pl.BlockSpec((BE, DE), lambda i: (i, 0)),
            pl.BlockSpec((D, DE), lambda i: (0, 0)),
            pl.BlockSpec((1, D), lambda i: (0, 0)),
        ],
        out_specs=pl.BlockSpec((BE, D), lambda i: (i, 0)),
        out_shape=jax.ShapeDtypeStruct((E, D), jnp.float32),
    )(edge_attr, We, be.reshape(1, D))


def _sc_aggregate(x, ef, idx_hbm):
    N, D = x.shape
    E = ef.shape[0]
    nch = E // _CH
    assert nch * _CH == E
    bcnt = nch // _NW
    extra = nch - bcnt * _NW
    ngrp = bcnt // _G
    assert ngrp * _G == bcnt
    nb = N // _CH
    nbr = N - nb * _CH
    nbpt = -(-nb // _NS)

    mesh = plsc.VectorSubcoreMesh(core_axis_name="c", subcore_axis_name="s")

    @functools.partial(
        pl.kernel,
        out_type=jax.ShapeDtypeStruct((_NC, N, D), jnp.float32),
        mesh=mesh,
        scratch_types=[
            pltpu.VMEM((_G, 1, _CH), jnp.int32),
            pltpu.VMEM((_G, 1, _CH), jnp.int32),
            pltpu.VMEM((2, _CH, D), jnp.float32),
            pltpu.VMEM((2, _CH, D), jnp.float32),
            pltpu.VMEM_SHARED((N, D), jnp.float32),
            pltpu.SemaphoreType.DMA((2,)),
            pltpu.SemaphoreType.DMA((2,)),
            pltpu.SemaphoreType.DMA((2,)),
            pltpu.SemaphoreType.DMA((2,)),
    ])
    def k(x_hbm, ef_hbm, idx, out_hbm,
          src_v, dst_v, xrows, efrows, agg_sh, semg, seme, semsx, semse):
        c = lax.axis_index("c")
        s = lax.axis_index("s")
        wid = s * _NC + c
        c0 = bcnt * wid + jnp.minimum(wid, extra)

        def zr(i, carry):
            xrows[0, i // (D // 16), pl.ds((i % (D // 16)) * 16, 16)] = (
                jnp.zeros((16,), jnp.float32))
            return carry
        lax.fori_loop(0, _CH * (D // 16), zr, 0)
        for t in range(nbpt):
            b = s + t * _NS

            @pl.when(b < nb)
            def _():
                pltpu.sync_copy(xrows.at[0], agg_sh.at[pl.ds(b * _CH, _CH)])
        if nbr:
            @pl.when(s == nb % _NS)
            def _():
                pltpu.sync_copy(xrows.at[0, pl.ds(0, nbr)],
                                agg_sh.at[pl.ds(nb * _CH, nbr)])
        plsc.subcore_barrier()

        def issue(ch, j, b):
            pltpu.async_copy(x_hbm.at[src_v.at[j, 0]], xrows.at[b],
                             semg.at[b])
            pltpu.async_copy(ef_hbm.at[pl.ds(ch * _CH, _CH)], efrows.at[b],
                             seme.at[b])

        def wait_in(j, b):
            pltpu.make_async_copy(x_hbm.at[src_v.at[j, 0]], xrows.at[b],
                                  semg.at[b]).wait()
            pltpu.make_async_copy(ef_hbm.at[pl.ds(0, _CH)], efrows.at[b],
                                  seme.at[b]).wait()

        def scat(j, b):
            pltpu.async_copy(xrows.at[b], agg_sh.at[dst_v.at[j, 0]],
                             semsx.at[b], add=True)
            pltpu.async_copy(efrows.at[b], agg_sh.at[dst_v.at[j, 0]],
                             semse.at[b], add=True)

        def wait_scat(j, b):
            pltpu.make_async_copy(xrows.at[b], agg_sh.at[dst_v.at[j, 0]],
                                  semsx.at[b]).wait()
            pltpu.make_async_copy(efrows.at[b], agg_sh.at[dst_v.at[j, 0]],
                                  semse.at[b]).wait()

        def group(g, carry):
            g0 = c0 + g * _G
            pltpu.sync_copy(idx.at[0, pl.ds(g0, _G)], src_v)
            pltpu.sync_copy(idx.at[1, pl.ds(g0, _G)], dst_v)
            issue(g0, 0, 0)

            def chunk(j, carry2):
                p = lax.rem(j, 2)
                q = 1 - p

                @pl.when(j + 1 < _G)
                def _():
                    @pl.when(j >= 1)
                    def _():
                        wait_scat(j - 1, q)
                    issue(g0 + j + 1, j + 1, q)
                wait_in(j, p)
                scat(j, p)
                return carry2
            lax.fori_loop(0, _G, chunk, 0)
            wait_scat(_G - 2, lax.rem(_G, 2))
            wait_scat(_G - 1, lax.rem(_G - 1, 2))
            return carry
        lax.fori_loop(0, ngrp, group, 0)

        if extra:
            @pl.when(wid < extra)
            def _():
                ce = c0 + bcnt
                pltpu.sync_copy(idx.at[0, pl.ds(ce, 1)],
                                src_v.at[pl.ds(0, 1)])
                pltpu.sync_copy(idx.at[1, pl.ds(ce, 1)],
                                dst_v.at[pl.ds(0, 1)])
                pltpu.sync_copy(x_hbm.at[src_v.at[0, 0]], xrows.at[0])
                pltpu.sync_copy(ef_hbm.at[pl.ds(ce * _CH, _CH)], efrows.at[0])
                pltpu.sync_copy(xrows.at[0], agg_sh.at[dst_v.at[0, 0]],
                                add=True)
                pltpu.sync_copy(efrows.at[0], agg_sh.at[dst_v.at[0, 0]],
                                add=True)

        plsc.subcore_barrier()
        for t in range(nbpt):
            b = s + t * _NS

            @pl.when(b < nb)
            def _():
                pltpu.sync_copy(agg_sh.at[pl.ds(b * _CH, _CH)],
                                out_hbm.at[c, pl.ds(b * _CH, _CH)])
        if nbr:
            @pl.when(s == nb % _NS)
            def _():
                pltpu.sync_copy(agg_sh.at[pl.ds(nb * _CH, nbr)],
                                out_hbm.at[c, pl.ds(nb * _CH, nbr)])

    return k(x, ef, idx_hbm)


def _mlp(x, agg, eps, W1, b1, g1, bt1, W2, b2, g2, bt2):
    N, D = x.shape

    def body(eps_ref, x_ref, a_ref, w1, b1r, g1r, t1r,
             w2, b2r, g2r, t2r, o):
        out = (1.0 + eps_ref[0, 0]) * x_ref[...] + a_ref[0] + a_ref[1]
        h = lax.dot_general(out, w1[...], _DN,
                            preferred_element_type=jnp.float32) + b1r[...]
        mu = jnp.mean(h, axis=0, keepdims=True)
        var = jnp.mean((h - mu) ** 2, axis=0, keepdims=True)
        h = jnp.maximum((h - mu) / jnp.sqrt(var + 1e-5) * g1r[...] + t1r[...],
                        0.0)
        h = lax.dot_general(h, w2[...], _DN,
                            preferred_element_type=jnp.float32) + b2r[...]
        mu = jnp.mean(h, axis=0, keepdims=True)
        var = jnp.mean((h - mu) ** 2, axis=0, keepdims=True)
        o[...] = jnp.maximum(
            (h - mu) / jnp.sqrt(var + 1e-5) * g2r[...] + t2r[...], 0.0)

    vspec = pl.BlockSpec(memory_space=pltpu.VMEM)
    return pl.pallas_call(
        body,
        in_specs=[pl.BlockSpec(memory_space=pltpu.SMEM)] + [vspec] * 10,
        out_specs=vspec,
        out_shape=jax.ShapeDtypeStruct((N, D), jnp.float32),
    )(eps.reshape(1, 1), x, agg, W1,
      b1.reshape(1, D), g1.reshape(1, D), bt1.reshape(1, D), W2,
      b2.reshape(1, D), g2.reshape(1, D), bt2.reshape(1, D))


def kernel(x, edge_index, edge_attr, epsilon, We, be,
           W1, b1, g1, bt1, W2, b2, g2, bt2):
    E = edge_index.shape[1]
    idx4 = edge_index.reshape(2, E // _CH, 1, _CH)
    ef = _edge_embed(edge_attr, We, be)
    agg = _sc_aggregate(x, ef, idx4)
    return _mlp(x, agg, epsilon, W1, b1, g1, bt1, W2, b2, g2, bt2)
